# baseline (speedup 1.0000x reference)
import functools

import jax
import jax.numpy as jnp
from jax import lax
from jax.experimental import pallas as pl
from jax.experimental.pallas import tpu as pltpu
from jax.experimental.pallas import tpu_sc as plsc
from jax.scipy.linalg import block_diag

B, N, C, NPOINTS = 4, 4096, 64, 32
E_TOTAL = N * NPOINTS
TCOLS = B * 32
NB = 128
CHUNK = 128
NPARTS = 2
NPRE = 1024


def _pre_body(xyz_ref, feats_ref, wx_ref, wf_ref, bias_ref, tab_ref, ctr_ref):
    dn0 = (((0,), (0,)), ((), ()))
    p = (lax.dot_general(xyz_ref[...].reshape(3 * B, NPRE), wx_ref[...], dn0,
                         preferred_element_type=jnp.float32)
         + lax.dot_general(feats_ref[...].reshape(B * C, NPRE), wf_ref[...],
                           dn0, preferred_element_type=jnp.float32)
         + bias_ref[...])
    tab_ref[...] = p[:, :TCOLS]
    ctr_ref[...] = p[:, TCOLS:]


def _tc_pre(xyzT, features, wx, wf, bias):
    return pl.pallas_call(
        _pre_body,
        grid=(N // NPRE,),
        in_specs=[
            pl.BlockSpec((3, B, NPRE), lambda i: (0, 0, i)),
            pl.BlockSpec((B, C, NPRE), lambda i: (0, 0, i)),
            pl.BlockSpec((3 * B, 2 * TCOLS), lambda i: (0, 0)),
            pl.BlockSpec((B * C, 2 * TCOLS), lambda i: (0, 0)),
            pl.BlockSpec((1, 2 * TCOLS), lambda i: (0, 0)),
        ],
        out_specs=[
            pl.BlockSpec((NPRE, TCOLS), lambda i: (i, 0)),
            pl.BlockSpec((NPRE, TCOLS), lambda i: (i, 0)),
        ],
        out_shape=[
            jax.ShapeDtypeStruct((N, TCOLS), jnp.float32),
            jax.ShapeDtypeStruct((N, TCOLS), jnp.float32),
        ],
    )(xyzT, features, wx, wf, bias)


def _sc_gather(tab, idx3, n_edges):
    info = plsc.get_sparse_core_info()
    nc, ns = info.num_cores, info.num_subcores
    nw = nc * ns
    per_w = n_edges // nw
    steps = per_w // CHUNK
    mesh = plsc.VectorSubcoreMesh(core_axis_name="c", subcore_axis_name="s")

    @functools.partial(
        pl.kernel, mesh=mesh,
        out_type=jax.ShapeDtypeStruct((n_edges, TCOLS), jnp.float32),
        scratch_types=[
            pltpu.VMEM((steps, CHUNK), jnp.int32),
            pltpu.VMEM((CHUNK, TCOLS), jnp.float32),
            pltpu.VMEM((CHUNK, TCOLS), jnp.float32),
            pltpu.SemaphoreType.DMA,
            pltpu.SemaphoreType.DMA,
        ],
    )
    def gk(tab_hbm, idx_hbm, out_hbm, idx_v, row0, row1, sem0, sem1):
        wid = lax.axis_index("s") * nc + lax.axis_index("c")
        base = wid * per_w
        pltpu.sync_copy(idx_hbm.at[wid], idx_v)

        def gather(t, row, sem):
            pltpu.async_copy(tab_hbm.at[idx_v.at[t]], row, sem)

        def wait(row, sem):
            pltpu.make_async_copy(tab_hbm.at[idx_v.at[0]], row, sem).wait()

        def scatter(t, row):
            off = pl.multiple_of(base + t * CHUNK, CHUNK)
            pltpu.sync_copy(row, out_hbm.at[pl.ds(off, CHUNK)])

        gather(0, row0, sem0)

        def body(g, carry):
            t0 = g * 2
            gather(t0 + 1, row1, sem1)
            wait(row0, sem0)
            scatter(t0, row0)

            @pl.when(g < steps // 2 - 1)
            def _():
                gather(t0 + 2, row0, sem0)

            wait(row1, sem1)
            scatter(t0 + 1, row1)
            return carry

        lax.fori_loop(0, steps // 2, body, 0)

    return gk(tab, idx3)


def _post_body(e_ref, ctr_ref, wh4_ref, bh4_ref, wf4_ref, bf4_ref,
               feat_ref, out_ref):
    e = e_ref[0]
    ctr = ctr_ref[0]
    uv = jax.nn.relu(e.reshape(NB, NPOINTS, TCOLS) + ctr[:, None, :])
    h = jax.nn.relu(
        lax.dot_general(uv.reshape(NB * NPOINTS, TCOLS).astype(jnp.bfloat16),
                        wh4_ref[...], (((1,), (0,)), ((), ())),
                        preferred_element_type=jnp.float32)
        + bh4_ref[...])
    m = jnp.mean(h.reshape(NB, NPOINTS, TCOLS), axis=1)
    o = jax.nn.relu(
        lax.dot_general(wf4_ref[...], m.astype(jnp.bfloat16),
                        (((1,), (1,)), ((), ())),
                        preferred_element_type=jnp.float32)
        + bf4_ref[...])
    out_ref[...] = o.reshape(B, 64, NB) + feat_ref[...]


def _tc_post(e4, ctr3, wh4, bh4, wf4, bf4, features, off, nblk):
    return pl.pallas_call(
        _post_body,
        grid=(nblk,),
        in_specs=[
            pl.BlockSpec((1, NB * NPOINTS, TCOLS), lambda i: (i, 0, 0)),
            pl.BlockSpec((1, NB, TCOLS), lambda i: (off + i, 0, 0)),
            pl.BlockSpec((TCOLS, TCOLS), lambda i: (0, 0)),
            pl.BlockSpec((1, TCOLS), lambda i: (0, 0)),
            pl.BlockSpec((B * 64, TCOLS), lambda i: (0, 0)),
            pl.BlockSpec((B * 64, 1), lambda i: (0, 0)),
            pl.BlockSpec((B, 64, NB), lambda i: (0, 0, off + i)),
        ],
        out_specs=pl.BlockSpec((B, 64, NB), lambda i: (0, 0, i)),
        out_shape=jax.ShapeDtypeStruct((B, 64, nblk * NB), jnp.float32),
    )(e4, ctr3, wh4, bh4, wf4, bf4, features)


def kernel(xyz, features, neighbor_idx, Wgu, bgu, Wgv, bgv, Wh, bh, Wf, bf):
    idx_flat = neighbor_idx.astype(jnp.int32).reshape(E_TOTAL)
    wx = jnp.zeros((3 * B, 2 * TCOLS), jnp.float32)
    wf_ = jnp.zeros((B * C, 2 * TCOLS), jnp.float32)
    bias = jnp.zeros((1, 2 * TCOLS), jnp.float32)
    for b in range(B):
        rows = jnp.arange(b, 3 * B, B)
        wx = wx.at[rows, 32 * b:32 * b + 16].set(Wgu[:, 3:].T)
        wx = wx.at[rows, TCOLS + 32 * b:TCOLS + 32 * b + 16].set(Wgu[:, :3].T)
        wf_ = wf_.at[C * b:C * (b + 1), 32 * b + 16:32 * b + 32].set(Wgv.T)
        wf_ = wf_.at[C * b:C * (b + 1),
                     TCOLS + 32 * b + 16:TCOLS + 32 * b + 32].set(Wgv.T)
        bias = bias.at[0, TCOLS + 32 * b:TCOLS + 32 * b + 16].set(bgu)
        bias = bias.at[0, TCOLS + 32 * b + 16:TCOLS + 32 * b + 32].set(bgv)
    xyzT = jnp.transpose(xyz, (2, 0, 1))
    tab, ctr = _tc_pre(xyzT, features, wx, wf_, bias)
    ctr3 = ctr.reshape(N // NB, NB, TCOLS)
    wh4 = block_diag(Wh.T, Wh.T, Wh.T, Wh.T).astype(jnp.bfloat16)
    bh4 = jnp.tile(bh, B).reshape(1, TCOLS)
    wf4 = block_diag(Wf, Wf, Wf, Wf).astype(jnp.bfloat16)
    bf4 = jnp.tile(bf, B).reshape(B * 64, 1)
    blk_split = (18, 14)
    outs = []
    off = 0
    for nblk in blk_split:
        ep = nblk * NB * NPOINTS
        idx3 = lax.dynamic_slice_in_dim(idx_flat, off * NB * NPOINTS, ep,
                                        0).reshape(32, ep // 32 // CHUNK, CHUNK)
        edges = _sc_gather(tab, idx3, ep)
        e4 = edges.reshape(nblk, NB * NPOINTS, TCOLS)
        outs.append(_tc_post(e4, ctr3, wh4, bh4, wf4, bf4, features,
                             off, nblk))
        off += nblk
    out = jnp.concatenate(outs, axis=2)
    return (xyz, out)


# speedup vs baseline: 2.2451x; 2.2451x over previous
import functools

import jax
import jax.numpy as jnp
from jax import lax
from jax.experimental import pallas as pl
from jax.experimental.pallas import tpu as pltpu
from jax.experimental.pallas import tpu_sc as plsc
from jax.scipy.linalg import block_diag

B, N, C, NPOINTS = 4, 4096, 64, 32
E_TOTAL = N * NPOINTS
TCOLS = B * 32
NB = 128
CHUNK = 128
NPARTS = 2
NPRE = 1024


def _pre_body(xyz_ref, feats_ref, wx_ref, wf_ref, bias_ref, tab_ref, ctr_ref):
    dn0 = (((0,), (0,)), ((), ()))
    p = (lax.dot_general(xyz_ref[...].reshape(3 * B, NPRE), wx_ref[...], dn0,
                         preferred_element_type=jnp.float32)
         + lax.dot_general(feats_ref[...].reshape(B * C, NPRE), wf_ref[...],
                           dn0, preferred_element_type=jnp.float32)
         + bias_ref[...])
    tab_ref[...] = p[:, :TCOLS]
    ctr_ref[...] = p[:, TCOLS:]


def _tc_pre(xyzT, features, wx, wf, bias):
    return pl.pallas_call(
        _pre_body,
        grid=(N // NPRE,),
        in_specs=[
            pl.BlockSpec((3, B, NPRE), lambda i: (0, 0, i)),
            pl.BlockSpec((B, C, NPRE), lambda i: (0, 0, i)),
            pl.BlockSpec((3 * B, 2 * TCOLS), lambda i: (0, 0)),
            pl.BlockSpec((B * C, 2 * TCOLS), lambda i: (0, 0)),
            pl.BlockSpec((1, 2 * TCOLS), lambda i: (0, 0)),
        ],
        out_specs=[
            pl.BlockSpec((NPRE, TCOLS), lambda i: (i, 0)),
            pl.BlockSpec((NPRE, TCOLS), lambda i: (i, 0)),
        ],
        out_shape=[
            jax.ShapeDtypeStruct((N, TCOLS), jnp.float32),
            jax.ShapeDtypeStruct((N, TCOLS), jnp.float32),
        ],
    )(xyzT, features, wx, wf, bias)


def _sc_gather(tab, idx3, n_edges):
    info = plsc.get_sparse_core_info()
    nc, ns = info.num_cores, info.num_subcores
    nw = nc * ns
    per_w = n_edges // nw
    steps = per_w // CHUNK
    mesh = plsc.VectorSubcoreMesh(core_axis_name="c", subcore_axis_name="s")

    @functools.partial(
        pl.kernel, mesh=mesh,
        out_type=jax.ShapeDtypeStruct((n_edges, TCOLS), jnp.float32),
        scratch_types=[
            pltpu.VMEM((steps, CHUNK), jnp.int32),
            pltpu.VMEM((CHUNK, TCOLS), jnp.float32),
            pltpu.VMEM((CHUNK, TCOLS), jnp.float32),
            pltpu.SemaphoreType.DMA,
            pltpu.SemaphoreType.DMA,
        ],
    )
    def gk(tab_hbm, idx_hbm, out_hbm, idx_v, row0, row1, sem0, sem1):
        wid = lax.axis_index("s") * nc + lax.axis_index("c")
        base = wid * per_w
        pltpu.sync_copy(idx_hbm.at[wid], idx_v)

        def gather(t, row, sem):
            pltpu.async_copy(tab_hbm.at[idx_v.at[t]], row, sem)

        def wait(row, sem):
            pltpu.make_async_copy(tab_hbm.at[idx_v.at[0]], row, sem).wait()

        def scatter(t, row):
            off = pl.multiple_of(base + t * CHUNK, CHUNK)
            pltpu.sync_copy(row, out_hbm.at[pl.ds(off, CHUNK)])

        gather(0, row0, sem0)

        def body(g, carry):
            t0 = g * 2
            gather(t0 + 1, row1, sem1)
            wait(row0, sem0)
            scatter(t0, row0)

            @pl.when(g < steps // 2 - 1)
            def _():
                gather(t0 + 2, row0, sem0)

            wait(row1, sem1)
            scatter(t0 + 1, row1)
            return carry

        lax.fori_loop(0, steps // 2, body, 0)

    return gk(tab, idx3)


def _post_body(e_ref, ctr_ref, wh4_ref, bh4_ref, wf4_ref, bf4_ref,
               feat_ref, out_ref):
    e = e_ref[0]
    ctr = ctr_ref[0]
    uv = jax.nn.relu(e.reshape(NB, NPOINTS, TCOLS) + ctr[:, None, :])
    h = jax.nn.relu(
        lax.dot_general(uv.reshape(NB * NPOINTS, TCOLS).astype(jnp.bfloat16),
                        wh4_ref[...], (((1,), (0,)), ((), ())),
                        preferred_element_type=jnp.float32)
        + bh4_ref[...])
    m = jnp.mean(h.reshape(NB, NPOINTS, TCOLS), axis=1)
    o = jax.nn.relu(
        lax.dot_general(wf4_ref[...], m.astype(jnp.bfloat16),
                        (((1,), (1,)), ((), ())),
                        preferred_element_type=jnp.float32)
        + bf4_ref[...])
    out_ref[...] = o.reshape(B, 64, NB) + feat_ref[...]


def _tc_post(e4, ctr3, wh4, bh4, wf4, bf4, features, off, nblk):
    return pl.pallas_call(
        _post_body,
        grid=(nblk,),
        in_specs=[
            pl.BlockSpec((1, NB * NPOINTS, TCOLS), lambda i: (i, 0, 0)),
            pl.BlockSpec((1, NB, TCOLS), lambda i: (off + i, 0, 0)),
            pl.BlockSpec((TCOLS, TCOLS), lambda i: (0, 0)),
            pl.BlockSpec((1, TCOLS), lambda i: (0, 0)),
            pl.BlockSpec((B * 64, TCOLS), lambda i: (0, 0)),
            pl.BlockSpec((B * 64, 1), lambda i: (0, 0)),
            pl.BlockSpec((B, 64, NB), lambda i: (0, 0, off + i)),
        ],
        out_specs=pl.BlockSpec((B, 64, NB), lambda i: (0, 0, i)),
        out_shape=jax.ShapeDtypeStruct((B, 64, nblk * NB), jnp.float32),
    )(e4, ctr3, wh4, bh4, wf4, bf4, features)


def kernel(xyz, features, neighbor_idx, Wgu, bgu, Wgv, bgv, Wh, bh, Wf, bf):
    idx_flat = neighbor_idx.astype(jnp.int32).reshape(E_TOTAL)
    z16 = jnp.zeros((1, 16), jnp.float32)
    wx_halves = []
    for wpart in (Wgu[:, 3:].T, Wgu[:, :3].T):
        rows = [block_diag(*([jnp.concatenate([wpart[d][None, :], z16],
                                              axis=1)] * B))
                for d in range(3)]
        wx_halves.append(jnp.stack(rows))
    wx = jnp.concatenate(wx_halves, axis=2).reshape(3 * B, 2 * TCOLS)
    gvp = jnp.concatenate([jnp.zeros((C, 16), jnp.float32), Wgv.T], axis=1)
    wf_half = block_diag(*([gvp] * B))
    wf_ = jnp.concatenate([wf_half, wf_half], axis=1)
    bias_half = jnp.tile(jnp.concatenate([bgu, bgv]), B)
    bias = jnp.concatenate([jnp.zeros((TCOLS,), jnp.float32),
                            bias_half]).reshape(1, 2 * TCOLS)
    xyzT = jnp.transpose(xyz, (2, 0, 1))
    tab, ctr = _tc_pre(xyzT, features, wx, wf_, bias)
    ctr3 = ctr.reshape(N // NB, NB, TCOLS)
    wh4 = block_diag(Wh.T, Wh.T, Wh.T, Wh.T).astype(jnp.bfloat16)
    bh4 = jnp.tile(bh, B).reshape(1, TCOLS)
    wf4 = block_diag(Wf, Wf, Wf, Wf).astype(jnp.bfloat16)
    bf4 = jnp.tile(bf, B).reshape(B * 64, 1)
    blk_split = (18, 14)
    outs = []
    off = 0
    for nblk in blk_split:
        ep = nblk * NB * NPOINTS
        idx3 = lax.dynamic_slice_in_dim(idx_flat, off * NB * NPOINTS, ep,
                                        0).reshape(32, ep // 32 // CHUNK, CHUNK)
        edges = _sc_gather(tab, idx3, ep)
        e4 = edges.reshape(nblk, NB * NPOINTS, TCOLS)
        outs.append(_tc_post(e4, ctr3, wh4, bh4, wf4, bf4, features,
                             off, nblk))
        off += nblk
    out = jnp.concatenate(outs, axis=2)
    return (xyz, out)
